# Initial kernel scaffold; baseline (speedup 1.0000x reference)
import jax
import jax.numpy as jnp
from jax.experimental import pallas as pl


def kernel(input_tensor, emb_weight):
    raise NotImplementedError("write your pallas kernel here")


# speedup vs baseline: 163.3229x; 163.3229x over previous
import functools

import jax
import jax.numpy as jnp
from jax import lax
from jax.experimental import pallas as pl
from jax.experimental.pallas import tpu as pltpu
from jax.experimental.pallas import tpu_sc as plsc

NUM_BINS_ = 256
L = 16
NC = 2
NS = 16
NW = NC * NS


def _sc_body(x_hbm, w_hbm, out_hbm, table_v, buf_v, per_w):
    wid = lax.axis_index("s") * NC + lax.axis_index("c")
    base = wid * per_w
    pltpu.sync_copy(w_hbm, table_v)
    pltpu.sync_copy(x_hbm.at[pl.ds(base, per_w)], buf_v)

    def body(i, carry):
        xv = buf_v[pl.ds(i * L, L)]
        xv = jnp.minimum(jnp.maximum(xv, 0.0), 1.0)
        idx = (xv * 255.0).astype(jnp.int32)
        wv = plsc.load_gather(table_v, [idx])
        center = (idx.astype(jnp.float32) + 0.5) * (1.0 / NUM_BINS_)
        buf_v[pl.ds(i * L, L)] = wv * center
        return carry

    lax.fori_loop(0, per_w // L, body, 0)
    pltpu.sync_copy(buf_v, out_hbm.at[pl.ds(base, per_w)])


def kernel(input_tensor, emb_weight):
    shape = input_tensor.shape
    n = input_tensor.size
    per_w = n // NW
    assert n % (NW * L) == 0

    x_flat = input_tensor.reshape(n)
    w_flat = emb_weight.reshape(-1)

    mesh = plsc.VectorSubcoreMesh(core_axis_name="c", subcore_axis_name="s")
    run = functools.partial(
        pl.kernel,
        mesh=mesh,
        out_type=jax.ShapeDtypeStruct((n,), jnp.float32),
        scratch_types=[
            pltpu.VMEM((NUM_BINS_,), jnp.float32),
            pltpu.VMEM((per_w,), jnp.float32),
        ],
        compiler_params=pltpu.CompilerParams(needs_layout_passes=False),
    )(functools.partial(_sc_body, per_w=per_w))
    out = run(x_flat, w_flat)
    return out.reshape(shape)


# speedup vs baseline: 284.5670x; 1.7424x over previous
import functools

import jax
import jax.numpy as jnp
from jax import lax
from jax.experimental import pallas as pl
from jax.experimental.pallas import tpu as pltpu
from jax.experimental.pallas import tpu_sc as plsc

NUM_BINS_ = 256
L = 16
NC = 2
NS = 16
NW = NC * NS


def _sc_body(x_hbm, w_hbm, out_hbm, table_v, buf_v, per_w):
    wid = lax.axis_index("s") * NC + lax.axis_index("c")
    base = wid * per_w
    pltpu.sync_copy(w_hbm, table_v)
    pltpu.sync_copy(x_hbm.at[pl.ds(base, per_w)], buf_v)

    @plsc.parallel_loop(0, per_w, L, unroll=8)
    def body(i):
        xv = buf_v[pl.ds(i, L)]
        xv = jnp.minimum(jnp.maximum(xv, 0.0), 1.0)
        idx = (xv * 255.0).astype(jnp.int32)
        wv = plsc.load_gather(table_v, [idx])
        center = (idx.astype(jnp.float32) + 0.5) * (1.0 / NUM_BINS_)
        buf_v[pl.ds(i, L)] = wv * center
    pltpu.sync_copy(buf_v, out_hbm.at[pl.ds(base, per_w)])


def kernel(input_tensor, emb_weight):
    shape = input_tensor.shape
    n = input_tensor.size
    per_w = n // NW
    assert n % (NW * L) == 0

    x_flat = input_tensor.reshape(n)
    w_flat = emb_weight.reshape(-1)

    mesh = plsc.VectorSubcoreMesh(core_axis_name="c", subcore_axis_name="s")
    run = functools.partial(
        pl.kernel,
        mesh=mesh,
        out_type=jax.ShapeDtypeStruct((n,), jnp.float32),
        scratch_types=[
            pltpu.VMEM((NUM_BINS_,), jnp.float32),
            pltpu.VMEM((per_w,), jnp.float32),
        ],
        compiler_params=pltpu.CompilerParams(needs_layout_passes=False),
    )(functools.partial(_sc_body, per_w=per_w))
    out = run(x_flat, w_flat)
    return out.reshape(shape)


# speedup vs baseline: 304.6133x; 1.0704x over previous
import functools

import jax
import jax.numpy as jnp
from jax import lax
from jax.experimental import pallas as pl
from jax.experimental.pallas import tpu as pltpu
from jax.experimental.pallas import tpu_sc as plsc

NUM_BINS_ = 256
L = 16
NC = 2
NS = 16
NW = NC * NS
NCHUNK = 8


def _sc_body(x_hbm, w_hbm, out_hbm, tw_v, tg_v, ib0, ib1, ob0, ob1,
             sem_i0, sem_i1, sem_o0, sem_o1, per_w):
    wid = lax.axis_index("s") * NC + lax.axis_index("c")
    base = wid * per_w
    chunk = per_w // NCHUNK

    ibufs = (ib0, ib1)
    obufs = (ob0, ob1)
    sem_i = (sem_i0, sem_i1)
    sem_o = (sem_o0, sem_o1)

    in_desc = [
        pltpu.async_copy(x_hbm.at[pl.ds(base + g * chunk, chunk)],
                         ibufs[g], sem_i[g])
        for g in range(2)
    ]
    pltpu.sync_copy(w_hbm, tw_v)

    @plsc.parallel_loop(0, NUM_BINS_, L)
    def _prep(k):
        kf = (lax.iota(jnp.int32, L) + k).astype(jnp.float32)
        tg_v[pl.ds(k, L)] = tw_v[pl.ds(k, L)] * ((kf + 0.5) * (1.0 / NUM_BINS_))

    out_desc = [None, None]
    for g in range(NCHUNK):
        s = g % 2
        ibuf, obuf = ibufs[s], obufs[s]
        in_desc[s].wait()
        if g >= 2:
            out_desc[s].wait()

        @plsc.parallel_loop(0, chunk, L, unroll=8)
        def _body(i):
            xv = ibuf[pl.ds(i, L)]
            xv = jnp.minimum(jnp.maximum(xv, 0.0), 1.0)
            idx = (xv * 255.0).astype(jnp.int32)
            obuf[pl.ds(i, L)] = plsc.load_gather(tg_v, [idx])

        out_desc[s] = pltpu.async_copy(
            obuf, out_hbm.at[pl.ds(base + g * chunk, chunk)], sem_o[s])
        if g + 2 < NCHUNK:
            in_desc[s] = pltpu.async_copy(
                x_hbm.at[pl.ds(base + (g + 2) * chunk, chunk)], ibuf, sem_i[s])
    out_desc[0].wait()
    out_desc[1].wait()


def kernel(input_tensor, emb_weight):
    shape = input_tensor.shape
    n = input_tensor.size
    per_w = n // NW
    chunk = per_w // NCHUNK
    assert n % (NW * NCHUNK * L) == 0

    x_flat = input_tensor.reshape(n)
    w_flat = emb_weight.reshape(-1)

    mesh = plsc.VectorSubcoreMesh(core_axis_name="c", subcore_axis_name="s")
    run = functools.partial(
        pl.kernel,
        mesh=mesh,
        out_type=jax.ShapeDtypeStruct((n,), jnp.float32),
        scratch_types=[
            pltpu.VMEM((NUM_BINS_,), jnp.float32),
            pltpu.VMEM((NUM_BINS_,), jnp.float32),
            pltpu.VMEM((chunk,), jnp.float32),
            pltpu.VMEM((chunk,), jnp.float32),
            pltpu.VMEM((chunk,), jnp.float32),
            pltpu.VMEM((chunk,), jnp.float32),
            pltpu.SemaphoreType.DMA,
            pltpu.SemaphoreType.DMA,
            pltpu.SemaphoreType.DMA,
            pltpu.SemaphoreType.DMA,
        ],
        compiler_params=pltpu.CompilerParams(needs_layout_passes=False),
    )(functools.partial(_sc_body, per_w=per_w))
    out = run(x_flat, w_flat)
    return out.reshape(shape)


# speedup vs baseline: 481.2115x; 1.5797x over previous
import functools

import jax
import jax.numpy as jnp
from jax import lax
from jax.experimental import pallas as pl
from jax.experimental.pallas import tpu as pltpu
from jax.experimental.pallas import tpu_sc as plsc

NUM_BINS_ = 256
L = 16
NC = 2
NS = 16
NW = NC * NS
NCHUNK = 8

_COL_OFFS = tuple(range(0, 192, L)) + (200 - L,)


def _sc_body(x_hbm, w_hbm, out_hbm, tw_v, tg_v, ib0, ib1, ob0, ob1,
             sem_i0, sem_i1, sem_o0, sem_o1, rows_w):
    wid = lax.axis_index("s") * NC + lax.axis_index("c")
    base = wid * rows_w
    crows = rows_w // NCHUNK

    ibufs = (ib0, ib1)
    obufs = (ob0, ob1)
    sem_i = (sem_i0, sem_i1)
    sem_o = (sem_o0, sem_o1)

    in_desc = [
        pltpu.async_copy(x_hbm.at[pl.ds(base + g * crows, crows), :],
                         ibufs[g], sem_i[g])
        for g in range(2)
    ]
    pltpu.sync_copy(w_hbm, tw_v)

    @plsc.parallel_loop(0, NUM_BINS_, L)
    def _prep(k):
        ids = lax.iota(jnp.int32, L) + k
        wv = plsc.load_gather(tw_v, [ids, jnp.zeros((L,), jnp.int32)])
        tg_v[pl.ds(k, L)] = wv * ((ids.astype(jnp.float32) + 0.5)
                                  * (1.0 / NUM_BINS_))

    out_desc = [None, None]
    for g in range(NCHUNK):
        s = g % 2
        ibuf, obuf = ibufs[s], obufs[s]
        in_desc[s].wait()
        if g >= 2:
            out_desc[s].wait()

        @plsc.parallel_loop(0, crows, 1, unroll=2)
        def _body(r):
            for c in _COL_OFFS:
                xv = ibuf[r, pl.ds(c, L)]
                xv = jnp.minimum(jnp.maximum(xv, 0.0), 1.0)
                idx = (xv * 255.0).astype(jnp.int32)
                obuf[r, pl.ds(c, L)] = plsc.load_gather(tg_v, [idx])

        out_desc[s] = pltpu.async_copy(
            obuf, out_hbm.at[pl.ds(base + g * crows, crows), :], sem_o[s])
        if g + 2 < NCHUNK:
            in_desc[s] = pltpu.async_copy(
                x_hbm.at[pl.ds(base + (g + 2) * crows, crows), :],
                ibuf, sem_i[s])
    out_desc[0].wait()
    out_desc[1].wait()


def kernel(input_tensor, emb_weight):
    rows, cols = input_tensor.shape
    rows_w = rows // NW
    crows = rows_w // NCHUNK
    assert rows % (NW * NCHUNK) == 0 and cols == 200

    mesh = plsc.VectorSubcoreMesh(core_axis_name="c", subcore_axis_name="s")
    run = functools.partial(
        pl.kernel,
        mesh=mesh,
        out_type=jax.ShapeDtypeStruct((rows, cols), jnp.float32),
        scratch_types=[
            pltpu.VMEM(emb_weight.shape, jnp.float32),
            pltpu.VMEM((NUM_BINS_,), jnp.float32),
            pltpu.VMEM((crows, cols), jnp.float32),
            pltpu.VMEM((crows, cols), jnp.float32),
            pltpu.VMEM((crows, cols), jnp.float32),
            pltpu.VMEM((crows, cols), jnp.float32),
            pltpu.SemaphoreType.DMA,
            pltpu.SemaphoreType.DMA,
            pltpu.SemaphoreType.DMA,
            pltpu.SemaphoreType.DMA,
        ],
        compiler_params=pltpu.CompilerParams(needs_layout_passes=False),
    )(functools.partial(_sc_body, rows_w=rows_w))
    return run(input_tensor, emb_weight)
